# baseline (speedup 1.0000x reference)
import jax
import jax.numpy as jnp
from jax.experimental import pallas as pl
from jax.experimental.pallas import tpu as pltpu

N, E, D, H = 10000, 160000, 256, 4
C = D // H
ETOT = E + N
EB = 1024
NEB = 167
EPAD = NEB * EB
NPAD = 10240
NB = 2048
NNB = NPAD // NB


def _proj_body(x_ref, wl_ref, bl_ref, wr_ref, br_ref, xl_ref, xr_ref):
    x = x_ref[...]
    xl_ref[...] = jnp.dot(x, wl_ref[...],
                          preferred_element_type=jnp.float32) + bl_ref[...]
    xr_ref[...] = jnp.dot(x, wr_ref[...],
                          preferred_element_type=jnp.float32) + br_ref[...]


@jax.jit
def _proj(x, Wl, bl, Wr, br):
    return pl.pallas_call(
        _proj_body,
        grid=(NNB,),
        in_specs=[
            pl.BlockSpec((NB, D), lambda i: (i, 0)),
            pl.BlockSpec((D, D), lambda i: (0, 0)),
            pl.BlockSpec((1, D), lambda i: (0, 0)),
            pl.BlockSpec((D, D), lambda i: (0, 0)),
            pl.BlockSpec((1, D), lambda i: (0, 0)),
        ],
        out_specs=[
            pl.BlockSpec((NB, D), lambda i: (i, 0)),
            pl.BlockSpec((NB, D), lambda i: (i, 0)),
        ],
        out_shape=[
            jax.ShapeDtypeStruct((NPAD, D), jnp.float32),
            jax.ShapeDtypeStruct((NPAD, D), jnp.float32),
        ],
    )(x, Wl, bl.reshape(1, D), Wr, br.reshape(1, D))


def _edge_body(srcc_ref, dstc_ref, att_ref, xl_ref, xr_ref,
               msg_ref, exb_ref, xls, xrs):
    e = pl.program_id(0)
    n = pl.program_id(1)
    lane = jax.lax.broadcasted_iota(jnp.int32, (EB, NB), 1) + n * NB
    oh_s = (srcc_ref[...] == lane).astype(jnp.bfloat16)
    oh_d = (dstc_ref[...] == lane).astype(jnp.bfloat16)

    @pl.when(n == 0)
    def _():
        xls[...] = jnp.zeros_like(xls)
        xrs[...] = jnp.zeros_like(xrs)

    xls[...] += jnp.dot(oh_s, xl_ref[...].astype(jnp.bfloat16),
                        preferred_element_type=jnp.float32)
    xrs[...] += jnp.dot(oh_d, xr_ref[...].astype(jnp.bfloat16),
                        preferred_element_type=jnp.float32)

    @pl.when(n == NNB - 1)
    def _():
        z = xls[...] + xrs[...]
        m = jnp.where(z >= 0, z, 0.2 * z) * att_ref[...]
        hh = jax.lax.broadcasted_iota(jnp.int32, (D, D), 0) // C
        ww = jax.lax.broadcasted_iota(jnp.int32, (D, D), 1) // C
        pp = (hh == ww).astype(jnp.float32)
        logits = jnp.dot(m, pp, preferred_element_type=jnp.float32)
        eid = e * EB + jax.lax.broadcasted_iota(jnp.int32, (EB, D), 0)
        ex = jnp.where(eid < ETOT, jnp.exp(logits), 0.0)
        msg_ref[...] = xls[...] * ex
        exb_ref[...] = ex


@jax.jit
def _edge(xl, xr, src_col, dst_col, attflat):
    return pl.pallas_call(
        _edge_body,
        grid=(NEB, NNB),
        in_specs=[
            pl.BlockSpec((EB, 1), lambda e, n: (e, 0)),
            pl.BlockSpec((EB, 1), lambda e, n: (e, 0)),
            pl.BlockSpec((1, D), lambda e, n: (0, 0)),
            pl.BlockSpec((NB, D), lambda e, n: (n, 0)),
            pl.BlockSpec((NB, D), lambda e, n: (n, 0)),
        ],
        out_specs=[
            pl.BlockSpec((EB, D), lambda e, n: (e, 0)),
            pl.BlockSpec((EB, D), lambda e, n: (e, 0)),
        ],
        out_shape=[
            jax.ShapeDtypeStruct((EPAD, D), jnp.float32),
            jax.ShapeDtypeStruct((EPAD, D), jnp.float32),
        ],
        scratch_shapes=[
            pltpu.VMEM((EB, D), jnp.float32),
            pltpu.VMEM((EB, D), jnp.float32),
        ],
    )(src_col, dst_col, attflat, xl, xr)


def _scat_body(dstc_ref, msg_ref, exb_ref, bias_ref, h_ref, g_ref, accs, dens):
    nblk = pl.program_id(0)
    e = pl.program_id(1)
    n0 = nblk * NB
    lane = jax.lax.broadcasted_iota(jnp.int32, (EB, NB), 1) + n0
    oh = (dstc_ref[...] == lane).astype(jnp.bfloat16)
    dn = (((0,), (0,)), ((), ()))

    @pl.when(e == 0)
    def _():
        accs[...] = jnp.zeros_like(accs)
        dens[...] = jnp.zeros_like(dens)

    accs[...] += jax.lax.dot_general(oh, msg_ref[...].astype(jnp.bfloat16),
                                     dn, preferred_element_type=jnp.float32)
    dens[...] += jax.lax.dot_general(oh, exb_ref[...].astype(jnp.bfloat16),
                                     dn, preferred_element_type=jnp.float32)

    @pl.when(e == NEB - 1)
    def _():
        out = accs[...] / (dens[...] + 1e-16) + bias_ref[...]
        h = jnp.maximum(out, 0.0)
        rid = n0 + jax.lax.broadcasted_iota(jnp.int32, (NB, D), 0)
        h = jnp.where(rid < N, h, 0.0)
        h_ref[...] = h
        gi = jnp.sum(h, axis=0, keepdims=True)
        g_ref[...] = jnp.where(nblk == 0, gi, g_ref[...] + gi)


@jax.jit
def _scat(msg, exb, dst_col, bias):
    return pl.pallas_call(
        _scat_body,
        grid=(NNB, NEB),
        in_specs=[
            pl.BlockSpec((EB, 1), lambda n, e: (e, 0)),
            pl.BlockSpec((EB, D), lambda n, e: (e, 0)),
            pl.BlockSpec((EB, D), lambda n, e: (e, 0)),
            pl.BlockSpec((1, D), lambda n, e: (0, 0)),
        ],
        out_specs=[
            pl.BlockSpec((NB, D), lambda n, e: (n, 0)),
            pl.BlockSpec((1, D), lambda n, e: (0, 0)),
        ],
        out_shape=[
            jax.ShapeDtypeStruct((NPAD, D), jnp.float32),
            jax.ShapeDtypeStruct((1, D), jnp.float32),
        ],
        scratch_shapes=[
            pltpu.VMEM((NB, D), jnp.float32),
            pltpu.VMEM((NB, D), jnp.float32),
        ],
    )(dst_col, msg, exb, bias.reshape(1, D))


def kernel(node_embeddings, edge_index, Wl1, bl1, Wr1, br1, att1, bias1,
           Wl2, bl2, Wr2, br2, att2, bias2):
    x = jnp.pad(node_embeddings, ((0, NPAD - N), (0, 0)))
    loops = jnp.arange(N, dtype=edge_index.dtype)
    pad = jnp.zeros((EPAD - ETOT,), dtype=edge_index.dtype)
    srcp = jnp.concatenate([edge_index[0], loops, pad])
    dstp = jnp.concatenate([edge_index[1], loops, pad])
    src_col = srcp.reshape(EPAD, 1)
    dst_col = dstp.reshape(EPAD, 1)

    for (Wl, bl, Wr, br, att, bias) in (
            (Wl1, bl1, Wr1, br1, att1, bias1),
            (Wl2, bl2, Wr2, br2, att2, bias2)):
        xl, xr = _proj(x, Wl, bl, Wr, br)
        msg, exb = _edge(xl, xr, src_col, dst_col, att.reshape(1, D))
        x, graph = _scat(msg, exb, dst_col, bias)
    return x[:N], graph


# speedup vs baseline: 1.0430x; 1.0430x over previous
import jax
import jax.numpy as jnp
from jax.experimental import pallas as pl
from jax.experimental.pallas import tpu as pltpu

N, E, D, H = 10000, 160000, 256, 4
C = D // H
ETOT = E + N
EB = 1024
NEB = 167
EPAD = NEB * EB
NPAD = 10240
NB = 2048
NNB = NPAD // NB


def _proj_body(x_ref, wl_ref, bl_ref, wr_ref, br_ref, xl_ref, xr_ref):
    x = x_ref[...]
    xl_ref[...] = jnp.dot(x, wl_ref[...],
                          preferred_element_type=jnp.float32) + bl_ref[...]
    xr_ref[...] = jnp.dot(x, wr_ref[...],
                          preferred_element_type=jnp.float32) + br_ref[...]


@jax.jit
def _proj(x, Wl, bl, Wr, br):
    return pl.pallas_call(
        _proj_body,
        grid=(NNB,),
        in_specs=[
            pl.BlockSpec((NB, D), lambda i: (i, 0)),
            pl.BlockSpec((D, D), lambda i: (0, 0)),
            pl.BlockSpec((1, D), lambda i: (0, 0)),
            pl.BlockSpec((D, D), lambda i: (0, 0)),
            pl.BlockSpec((1, D), lambda i: (0, 0)),
        ],
        out_specs=[
            pl.BlockSpec((NB, D), lambda i: (i, 0)),
            pl.BlockSpec((NB, D), lambda i: (i, 0)),
        ],
        out_shape=[
            jax.ShapeDtypeStruct((NPAD, D), jnp.float32),
            jax.ShapeDtypeStruct((NPAD, D), jnp.float32),
        ],
    )(x, Wl, bl.reshape(1, D), Wr, br.reshape(1, D))


def _edge_body(srcc_ref, dstc_ref, att_ref, xl_ref, xr_ref,
               msg_ref, exb_ref, xls, xrs):
    e = pl.program_id(0)
    n = pl.program_id(1)
    lane = jax.lax.broadcasted_iota(jnp.int32, (EB, NB), 1) + n * NB
    oh_s = (srcc_ref[...] == lane).astype(jnp.float32)
    oh_d = (dstc_ref[...] == lane).astype(jnp.float32)

    @pl.when(n == 0)
    def _():
        xls[...] = jnp.zeros_like(xls)
        xrs[...] = jnp.zeros_like(xrs)

    xls[...] += jnp.dot(oh_s, xl_ref[...], preferred_element_type=jnp.float32)
    xrs[...] += jnp.dot(oh_d, xr_ref[...], preferred_element_type=jnp.float32)

    @pl.when(n == NNB - 1)
    def _():
        z = xls[...] + xrs[...]
        m = jnp.where(z >= 0, z, 0.2 * z) * att_ref[...]
        hh = jax.lax.broadcasted_iota(jnp.int32, (D, D), 0) // C
        ww = jax.lax.broadcasted_iota(jnp.int32, (D, D), 1) // C
        pp = (hh == ww).astype(jnp.float32)
        logits = jnp.dot(m, pp, preferred_element_type=jnp.float32)
        eid = e * EB + jax.lax.broadcasted_iota(jnp.int32, (EB, D), 0)
        ex = jnp.where(eid < ETOT, jnp.exp(logits), 0.0)
        msg_ref[...] = xls[...] * ex
        h2 = jax.lax.broadcasted_iota(jnp.int32, (D, D // 2), 0) // C
        w2 = jax.lax.broadcasted_iota(jnp.int32, (D, D // 2), 1) // (C // 2)
        pp2 = (h2 == w2).astype(jnp.float32)
        logits2 = jnp.dot(m, pp2, preferred_element_type=jnp.float32)
        eid2 = e * EB + jax.lax.broadcasted_iota(jnp.int32, (EB, D // 2), 0)
        exb_ref[...] = jnp.where(eid2 < ETOT, jnp.exp(logits2), 0.0)


@jax.jit
def _edge(xl, xr, src_col, dst_col, attflat):
    return pl.pallas_call(
        _edge_body,
        grid=(NEB, NNB),
        in_specs=[
            pl.BlockSpec((EB, 1), lambda e, n: (e, 0)),
            pl.BlockSpec((EB, 1), lambda e, n: (e, 0)),
            pl.BlockSpec((1, D), lambda e, n: (0, 0)),
            pl.BlockSpec((NB, D), lambda e, n: (n, 0)),
            pl.BlockSpec((NB, D), lambda e, n: (n, 0)),
        ],
        out_specs=[
            pl.BlockSpec((EB, D), lambda e, n: (e, 0)),
            pl.BlockSpec((EB, D // 2), lambda e, n: (e, 0)),
        ],
        out_shape=[
            jax.ShapeDtypeStruct((EPAD, D), jnp.float32),
            jax.ShapeDtypeStruct((EPAD, D // 2), jnp.float32),
        ],
        scratch_shapes=[
            pltpu.VMEM((EB, D), jnp.float32),
            pltpu.VMEM((EB, D), jnp.float32),
        ],
    )(src_col, dst_col, attflat, xl, xr)


def _scat_body(dstc_ref, msg_ref, exb_ref, bias_ref, h_ref, g_ref, accs, dens):
    nblk = pl.program_id(0)
    e = pl.program_id(1)
    n0 = nblk * NB
    lane = jax.lax.broadcasted_iota(jnp.int32, (EB, NB), 1) + n0
    oh = (dstc_ref[...] == lane).astype(jnp.float32)
    dn = (((0,), (0,)), ((), ()))

    @pl.when(e == 0)
    def _():
        accs[...] = jnp.zeros_like(accs)
        dens[...] = jnp.zeros_like(dens)

    accs[...] += jax.lax.dot_general(oh, msg_ref[...], dn,
                                     preferred_element_type=jnp.float32)
    dens[...] += jax.lax.dot_general(oh, exb_ref[...], dn,
                                     preferred_element_type=jnp.float32)

    @pl.when(e == NEB - 1)
    def _():
        q2a = jax.lax.broadcasted_iota(jnp.int32, (D // 2, D), 0) // (C // 2)
        q2b = jax.lax.broadcasted_iota(jnp.int32, (D // 2, D), 1) // C
        q2 = (q2a == q2b).astype(jnp.float32)
        den = jnp.dot(dens[...], q2,
                      preferred_element_type=jnp.float32) * 0.03125
        out = accs[...] / (den + 1e-16) + bias_ref[...]
        h = jnp.maximum(out, 0.0)
        rid = n0 + jax.lax.broadcasted_iota(jnp.int32, (NB, D), 0)
        h = jnp.where(rid < N, h, 0.0)
        h_ref[...] = h
        gi = jnp.sum(h, axis=0, keepdims=True)
        g_ref[...] = jnp.where(nblk == 0, gi, g_ref[...] + gi)


@jax.jit
def _scat(msg, exb, dst_col, bias):
    return pl.pallas_call(
        _scat_body,
        grid=(NNB, NEB),
        in_specs=[
            pl.BlockSpec((EB, 1), lambda n, e: (e, 0)),
            pl.BlockSpec((EB, D), lambda n, e: (e, 0)),
            pl.BlockSpec((EB, D // 2), lambda n, e: (e, 0)),
            pl.BlockSpec((1, D), lambda n, e: (0, 0)),
        ],
        out_specs=[
            pl.BlockSpec((NB, D), lambda n, e: (n, 0)),
            pl.BlockSpec((1, D), lambda n, e: (0, 0)),
        ],
        out_shape=[
            jax.ShapeDtypeStruct((NPAD, D), jnp.float32),
            jax.ShapeDtypeStruct((1, D), jnp.float32),
        ],
        scratch_shapes=[
            pltpu.VMEM((NB, D), jnp.float32),
            pltpu.VMEM((NB, D // 2), jnp.float32),
        ],
    )(dst_col, msg, exb, bias.reshape(1, D))


def kernel(node_embeddings, edge_index, Wl1, bl1, Wr1, br1, att1, bias1,
           Wl2, bl2, Wr2, br2, att2, bias2):
    x = jnp.pad(node_embeddings, ((0, NPAD - N), (0, 0)))
    loops = jnp.arange(N, dtype=edge_index.dtype)
    pad = jnp.zeros((EPAD - ETOT,), dtype=edge_index.dtype)
    srcp = jnp.concatenate([edge_index[0], loops, pad])
    dstp = jnp.concatenate([edge_index[1], loops, pad])
    src_col = srcp.reshape(EPAD, 1)
    dst_col = dstp.reshape(EPAD, 1)

    for (Wl, bl, Wr, br, att, bias) in (
            (Wl1, bl1, Wr1, br1, att1, bias1),
            (Wl2, bl2, Wr2, br2, att2, bias2)):
        xl, xr = _proj(x, Wl, bl, Wr, br)
        msg, exb = _edge(xl, xr, src_col, dst_col, att.reshape(1, D))
        x, graph = _scat(msg, exb, dst_col, bias)
    return x[:N], graph
